# Initial kernel scaffold; baseline (speedup 1.0000x reference)
import jax
import jax.numpy as jnp
from jax.experimental import pallas as pl


def kernel(phoneme_flat, phoneme_token_len, table):
    raise NotImplementedError("write your pallas kernel here")


# speedup vs baseline: 8.2471x; 8.2471x over previous
import functools

import jax
import jax.numpy as jnp
from jax import lax
from jax.experimental import pallas as pl
from jax.experimental.pallas import tpu as pltpu
from jax.experimental.pallas import tpu_sc as plsc

_NC = 2
_NS = 16
_NW = _NC * _NS
_LANES = 16
_T = 8


def _compose_body(nt, d, tpw, nsamp, idx_hbm, len_hbm, table_hbm, out_hbm, mask_hbm,
                  idx_all, idx_chunk, rows_v, out_v, mask_v, len_v, sem):
    cpw = tpw // _T
    cid = lax.axis_index("c")
    sid = lax.axis_index("s")
    wid = sid * _NC + cid
    g0 = wid * tpw

    pltpu.sync_copy(len_hbm, len_v)
    pltpu.sync_copy(idx_hbm.at[pl.ds(g0 * 4, tpw * 4)], idx_all)

    wpersamp = _NW // nsamp
    b = wid // wpersamp
    r0 = (wid % wpersamp) * tpw
    lens_vec = len_v[...]
    lane = lax.iota(jnp.int32, _LANES)
    lb = jnp.max(jnp.where(lane == b, lens_vec, 0))
    nv = jnp.minimum(jnp.maximum(lb - r0, 0), tpw)

    def mask_grp(grp, carry):
        t = lax.iota(jnp.int32, _LANES) + grp * _LANES
        p = t * 4
        v = plsc.load_gather(idx_all, [p])
        for j in range(1, 4):
            v = v | plsc.load_gather(idx_all, [p + j])
        m = ((v != 0) & (t < nv)).astype(jnp.int32)
        mask_v[pl.ds(pl.multiple_of(grp * _LANES, _LANES), _LANES)] = m
        return carry

    lax.fori_loop(0, tpw // _LANES, mask_grp, 0)
    pltpu.sync_copy(mask_v, mask_hbm.at[pl.ds(g0, tpw)])

    nchunks = (nv + _T - 1) // _T

    def chunk_body(cix, carry):
        base = cix * (4 * _T)
        for h in range(4 * _T // _LANES):
            lane = lax.iota(jnp.int32, _LANES)
            tok = ((lane + h * _LANES) >> 2) + cix * _T
            v = idx_all[pl.ds(pl.multiple_of(base + h * _LANES, _LANES), _LANES)]
            v = jnp.where(tok < nv, v, 0)
            idx_chunk[pl.ds(h * _LANES, _LANES)] = v
        pltpu.async_copy(table_hbm.at[idx_chunk], rows_v, sem).wait()
        for t in range(_T):
            def dbody(dd, c2, t=t):
                sl = pl.ds(pl.multiple_of(dd * _LANES, _LANES), _LANES)
                a = rows_v[4 * t, sl] + rows_v[4 * t + 1, sl]
                bb = rows_v[4 * t + 2, sl] + rows_v[4 * t + 3, sl]
                out_v[t, sl] = a + bb
                return c2
            lax.fori_loop(0, d // _LANES, dbody, 0)
        pltpu.sync_copy(out_v, out_hbm.at[pl.ds(g0 + cix * _T, _T), :])
        return carry

    lax.fori_loop(0, nchunks, chunk_body, 0)

    zeros = jnp.zeros((_LANES,), jnp.float32)
    for t in range(_T):
        def zbody(dd, c2, t=t):
            sl = pl.ds(pl.multiple_of(dd * _LANES, _LANES), _LANES)
            out_v[t, sl] = zeros
            return c2
        lax.fori_loop(0, d // _LANES, zbody, 0)

    def zfill(cix, carry):
        pltpu.sync_copy(out_v, out_hbm.at[pl.ds(g0 + cix * _T, _T), :])
        return carry

    lax.fori_loop(nchunks, cpw, zfill, 0)


@functools.partial(jax.jit, static_argnames=("nt", "d", "nsamp"))
def _compose_sc(idx_flat, len_pad, table, *, nt, d, nsamp):
    tpw = nt // _NW
    mesh = plsc.VectorSubcoreMesh(
        core_axis_name="c", subcore_axis_name="s",
        num_cores=_NC, num_subcores=_NS)
    body = functools.partial(_compose_body, nt, d, tpw, nsamp)
    return pl.kernel(
        body,
        out_type=[
            jax.ShapeDtypeStruct((nt, d), jnp.float32),
            jax.ShapeDtypeStruct((nt,), jnp.int32),
        ],
        mesh=mesh,
        compiler_params=pltpu.CompilerParams(needs_layout_passes=False),
        scratch_types=[
            pltpu.VMEM((tpw * 4,), jnp.int32),
            pltpu.VMEM((4 * _T,), jnp.int32),
            pltpu.VMEM((4 * _T, d), jnp.float32),
            pltpu.VMEM((_T, d), jnp.float32),
            pltpu.VMEM((tpw,), jnp.int32),
            pltpu.VMEM((_LANES,), jnp.int32),
            pltpu.SemaphoreType.DMA,
        ],
    )(idx_flat, len_pad, table)


def kernel(phoneme_flat, phoneme_token_len, table):
    bsz, pt = phoneme_flat.shape
    lx = pt // 4
    nt = bsz * lx
    d = table.shape[1]
    idx_flat = phoneme_flat.reshape(-1).astype(jnp.int32)
    len_pad = jnp.zeros((_LANES,), jnp.int32).at[:bsz].set(
        phoneme_token_len.astype(jnp.int32))
    out_flat, mask_i = _compose_sc(idx_flat, len_pad, table, nt=nt, d=d,
                                   nsamp=bsz)
    out = out_flat.reshape(bsz, lx, d)
    pf_mask = mask_i.reshape(bsz, lx).astype(bool)
    return out, pf_mask


# speedup vs baseline: 18.5105x; 2.2445x over previous
import functools

import jax
import jax.numpy as jnp
from jax import lax
from jax.experimental import pallas as pl
from jax.experimental.pallas import tpu as pltpu
from jax.experimental.pallas import tpu_sc as plsc

_NC = 2
_NS = 16
_NW = _NC * _NS
_LANES = 16
_T = 8


def _compose_body(nt, d, tpw, nsamp, idx_hbm, len_hbm, table_hbm, out_hbm,
                  mask_hbm, idx_all, mask_v, len_v, ibufs, gbufs, rows, obufs,
                  isem, gsem, osem):
    nslots = tpw // _T
    lsz = nt // nsamp
    cid = lax.axis_index("c")
    sid = lax.axis_index("s")
    wid = sid * _NC + cid
    g0 = wid * tpw

    pltpu.sync_copy(len_hbm, len_v)
    pltpu.sync_copy(idx_hbm.at[pl.ds(g0 * 4, tpw * 4)], idx_all)
    lens_vec = len_v[...]
    lane = lax.iota(jnp.int32, _LANES)

    wpersamp = _NW // nsamp
    b = wid // wpersamp
    r0 = (wid % wpersamp) * tpw
    lb = jnp.max(jnp.where(lane == b, lens_vec, 0))
    nv = jnp.minimum(jnp.maximum(lb - r0, 0), tpw)

    def mask_grp(grp, carry):
        t = lane + grp * _LANES
        p = t * 4
        v = plsc.load_gather(idx_all, [p])
        for j in range(1, 4):
            v = v | plsc.load_gather(idx_all, [p + j])
        m = ((v != 0) & (t < nv)).astype(jnp.int32)
        mask_v[pl.ds(pl.multiple_of(grp * _LANES, _LANES), _LANES)] = m
        return carry

    lax.fori_loop(0, tpw // _LANES, mask_grp, 0)
    pltpu.sync_copy(mask_v, mask_hbm.at[pl.ds(g0, tpw)])

    def slot_info(j):
        gq = (wid + _NW * j) * _T
        bq = gq // lsz
        l0 = gq - bq * lsz
        lbq = jnp.max(jnp.where(lane == bq, lens_vec, 0))
        nvq = jnp.minimum(jnp.maximum(lbq - l0, 0), _T)
        return gq, nvq

    def fire_idx(j, ib):
        gq = (wid + _NW * j) * _T
        pltpu.async_copy(idx_hbm.at[pl.ds(gq * 4, 4 * _T)], ib, isem)

    def wait_idx(ib):
        pltpu.make_async_copy(idx_hbm.at[pl.ds(0, 4 * _T)], ib, isem).wait()

    def prep_gather(ib, gb, rws, nvq):
        for h in range(4 * _T // _LANES):
            tok = (lane >> 2) + 4 * h
            v = ib[pl.ds(h * _LANES, _LANES)]
            gb[pl.ds(h * _LANES, _LANES)] = jnp.where(tok < nvq, v, 0)

        @pl.when(nvq > 0)
        def _():
            pltpu.async_copy(table_hbm.at[gb], rws, gsem)

    def wait_gather(gb, rws, nvq):
        @pl.when(nvq > 0)
        def _():
            pltpu.make_async_copy(table_hbm.at[gb], rws, gsem).wait()

    def compute_out(rws, ob, nvq):
        @pl.when(nvq > 0)
        def _():
            for t in range(_T):
                def dbody(dd, c, t=t):
                    sl = pl.ds(pl.multiple_of(dd * _LANES, _LANES), _LANES)
                    ob[t, sl] = ((rws[4 * t, sl] + rws[4 * t + 1, sl]) +
                                 (rws[4 * t + 2, sl] + rws[4 * t + 3, sl]))
                    return c
                lax.fori_loop(0, d // _LANES, dbody, 0, unroll=4)

        @pl.when(nvq == 0)
        def _():
            zeros = jnp.zeros((_LANES,), jnp.float32)
            for t in range(_T):
                def zbody(dd, c, t=t):
                    sl = pl.ds(pl.multiple_of(dd * _LANES, _LANES), _LANES)
                    ob[t, sl] = zeros
                    return c
                lax.fori_loop(0, d // _LANES, zbody, 0, unroll=4)

    def fire_out(j, ob):
        gq = (wid + _NW * j) * _T
        pltpu.async_copy(ob, out_hbm.at[pl.ds(gq, _T), :], osem)

    def wait_out(ob):
        pltpu.make_async_copy(ob, out_hbm.at[pl.ds(0, _T), :], osem).wait()

    fire_idx(0, ibufs[0])
    fire_idx(1, ibufs[1])
    _, nv0 = slot_info(0)
    wait_idx(ibufs[0])
    prep_gather(ibufs[0], gbufs[0], rows[0], nv0)

    npairs = nslots // 2

    def pair_body(i, carry):
        a = 2 * i
        bslot = a + 1
        _, nva = slot_info(a)
        _, nvb = slot_info(bslot)

        @pl.when(i < npairs - 1)
        def _():
            fire_idx(a + 2, ibufs[0])
        wait_idx(ibufs[1])
        prep_gather(ibufs[1], gbufs[1], rows[1], nvb)
        wait_gather(gbufs[0], rows[0], nva)

        @pl.when(i >= 1)
        def _():
            wait_out(obufs[0])
        compute_out(rows[0], obufs[0], nva)
        fire_out(a, obufs[0])

        @pl.when(i < npairs - 1)
        def _():
            fire_idx(bslot + 2, ibufs[1])
            wait_idx(ibufs[0])
            _, nvn = slot_info(a + 2)
            prep_gather(ibufs[0], gbufs[0], rows[0], nvn)
        wait_gather(gbufs[1], rows[1], nvb)

        @pl.when(i >= 1)
        def _():
            wait_out(obufs[1])
        compute_out(rows[1], obufs[1], nvb)
        fire_out(bslot, obufs[1])
        return carry

    lax.fori_loop(0, npairs, pair_body, 0)
    wait_out(obufs[0])
    wait_out(obufs[1])


@functools.partial(jax.jit, static_argnames=("nt", "d", "nsamp"))
def _compose_sc(idx_flat, len_pad, table, *, nt, d, nsamp):
    tpw = nt // _NW
    mesh = plsc.VectorSubcoreMesh(
        core_axis_name="c", subcore_axis_name="s",
        num_cores=_NC, num_subcores=_NS)
    body = functools.partial(_compose_body, nt, d, tpw, nsamp)
    return pl.kernel(
        body,
        out_type=[
            jax.ShapeDtypeStruct((nt, d), jnp.float32),
            jax.ShapeDtypeStruct((nt,), jnp.int32),
        ],
        mesh=mesh,
        compiler_params=pltpu.CompilerParams(needs_layout_passes=False),
        scratch_types=[
            pltpu.VMEM((tpw * 4,), jnp.int32),
            pltpu.VMEM((tpw,), jnp.int32),
            pltpu.VMEM((_LANES,), jnp.int32),
            [pltpu.VMEM((4 * _T,), jnp.int32)] * 2,
            [pltpu.VMEM((4 * _T,), jnp.int32)] * 2,
            [pltpu.VMEM((4 * _T, d), jnp.float32)] * 2,
            [pltpu.VMEM((_T, d), jnp.float32)] * 2,
            pltpu.SemaphoreType.DMA,
            pltpu.SemaphoreType.DMA,
            pltpu.SemaphoreType.DMA,
        ],
    )(idx_flat, len_pad, table)


def kernel(phoneme_flat, phoneme_token_len, table):
    bsz, pt = phoneme_flat.shape
    lx = pt // 4
    nt = bsz * lx
    d = table.shape[1]
    idx_flat = phoneme_flat.reshape(-1).astype(jnp.int32)
    len_pad = jnp.zeros((_LANES,), jnp.int32).at[:bsz].set(
        phoneme_token_len.astype(jnp.int32))
    out_flat, mask_i = _compose_sc(idx_flat, len_pad, table, nt=nt, d=d,
                                   nsamp=bsz)
    out = out_flat.reshape(bsz, lx, d)
    pf_mask = mask_i.reshape(bsz, lx).astype(bool)
    return out, pf_mask
